# baseline (speedup 1.0000x reference)
import functools

import jax
import jax.numpy as jnp
from jax import lax
from jax.experimental import pallas as pl
from jax.experimental.pallas import tpu as pltpu
from jax.experimental.pallas import tpu_sc as plsc

N_EDGES = 1600000
EMBED = 32
NUM_ROWS = 1000
NUM_CORES = 2
NUM_SUBCORES = 16
NUM_WORKERS = NUM_CORES * NUM_SUBCORES
PER_WORKER = N_EDGES // NUM_WORKERS
CHUNK = 400
N_CHUNKS = PER_WORKER // CHUNK
GROUPS = CHUNK // 16
NBUF = 2

_GDN = lax.GatherDimensionNumbers(
    offset_dims=(), collapsed_slice_dims=(0,), start_index_map=(0,))


def _bcast_lane(vec, k):
    kv = jnp.full((16, 1), k, jnp.int32)
    return lax.gather(vec, kv, _GDN, (1,),
                      mode=lax.GatherScatterMode.PROMISE_IN_BOUNDS)


def _body(idx_hbm, table_hbm, out_hbm, idx_v, rows_v, tbl_v, sem_t, sem_i,
          sem_o):
    wid = lax.axis_index("s") * NUM_CORES + lax.axis_index("c")
    base = wid * PER_WORKER

    pltpu.async_copy(table_hbm, tbl_v, sem_t).wait()

    iota16 = lax.iota(jnp.int32, 16)

    def idx_cp(g, b):
        return pltpu.make_async_copy(
            idx_hbm.at[pl.ds(base + g * CHUNK, CHUNK)], idx_v.at[b], sem_i.at[b])

    def out_cp(g, b):
        return pltpu.make_async_copy(
            rows_v.at[b],
            out_hbm.at[pl.ds((base + g * CHUNK) * EMBED, CHUNK * EMBED)],
            sem_o.at[b])

    for b in range(NBUF):
        idx_cp(b, b).start()

    @pl.loop(0, N_CHUNKS)
    def step(g):
        b = lax.rem(g, NBUF)

        @pl.when(g >= NBUF)
        def _():
            out_cp(g - NBUF, b).wait()

        idx_cp(g, b).wait()
        rows = rows_v.at[b]
        idxc = idx_v.at[b]

        @pl.loop(0, GROUPS)
        def grp(j):
            ridx = idxc[pl.ds(j * 16, 16)]
            addr = lax.shift_left(ridx, jnp.int32(5))
            for k in range(16):
                a0 = _bcast_lane(addr, k) + iota16
                v0 = plsc.load_gather(tbl_v, [a0])
                v1 = plsc.load_gather(tbl_v, [a0 + 16])
                off = (j * 16 + k) * EMBED
                rows[pl.ds(off, 16)] = v0
                rows[pl.ds(off + 16, 16)] = v1

        @pl.when(g + NBUF < N_CHUNKS)
        def _():
            idx_cp(g + NBUF, b).start()

        out_cp(g, b).start()

    for k in range(NBUF):
        g = N_CHUNKS - NBUF + k
        out_cp(g, g % NBUF).wait()


@functools.partial(jax.jit, static_argnames=())
def kernel(edge_type, table):
    idx = edge_type.astype(jnp.int32)
    tbl_flat = table.reshape(-1)
    mesh = plsc.VectorSubcoreMesh(
        core_axis_name="c", subcore_axis_name="s", num_cores=NUM_CORES
    )
    run = pl.kernel(
        _body,
        out_type=jax.ShapeDtypeStruct((N_EDGES * EMBED,), jnp.float32),
        mesh=mesh,
        scratch_types=[
            pltpu.VMEM((NBUF, CHUNK), jnp.int32),
            pltpu.VMEM((NBUF, CHUNK * EMBED), jnp.float32),
            pltpu.VMEM((NUM_ROWS * EMBED,), jnp.float32),
            pltpu.SemaphoreType.DMA,
            pltpu.SemaphoreType.DMA((NBUF,)),
            pltpu.SemaphoreType.DMA((NBUF,)),
        ],
        compiler_params=pltpu.CompilerParams(use_tc_tiling_on_sc=False,
                                             needs_layout_passes=False),
    )
    out_flat = run(idx, tbl_flat)
    return out_flat.reshape(N_EDGES, EMBED)


# speedup vs baseline: 2.3456x; 2.3456x over previous
import functools

import jax
import jax.numpy as jnp
from jax import lax
from jax.experimental import pallas as pl
from jax.experimental.pallas import tpu as pltpu
from jax.experimental.pallas import tpu_sc as plsc

N_EDGES = 1600000
EMBED = 32
NUM_ROWS = 1000
NUM_CORES = 2
NUM_SUBCORES = 16
NUM_WORKERS = NUM_CORES * NUM_SUBCORES
BLK = 128
N_BLOCKS = N_EDGES // BLK
BASE_BLOCKS = N_BLOCKS // NUM_WORKERS
EXTRA = N_BLOCKS - BASE_BLOCKS * NUM_WORKERS
MAXBLK = BASE_BLOCKS + 1


def _body(idx_hbm, tbl_hbm, out_hbm, idx_v, tbl_v, buf_v, sem_t, sem_i, sem_o):
    wid = lax.axis_index("s") * NUM_CORES + lax.axis_index("c")
    nblk = jnp.where(wid < EXTRA, BASE_BLOCKS + 1, BASE_BLOCKS)
    blk0 = wid * BASE_BLOCKS + jnp.minimum(wid, EXTRA)

    pltpu.async_copy(tbl_hbm, tbl_v, sem_t).wait()

    col_const = [jnp.full((16,), d * NUM_ROWS, jnp.int32) for d in range(EMBED)]

    def idx_cp(j, b):
        return pltpu.make_async_copy(
            idx_hbm.at[pl.ds((blk0 + j) * BLK, BLK)],
            idx_v.at[pl.ds(b * BLK, BLK)], sem_i.at[b])

    def out_cp(j, b, r):
        return pltpu.make_async_copy(
            buf_v.at[b * 4 + r],
            out_hbm.at[pl.ds(8 * r, 8), pl.ds((blk0 + j) * BLK, BLK)],
            sem_o.at[b])

    idx_cp(0, 0).start()

    @pl.loop(0, MAXBLK)
    def step(j):
        b = lax.rem(j, 2)

        @pl.when(j < nblk)
        def _():
            idx_cp(j, b).wait()

            @pl.when(j + 1 < nblk)
            def _():
                idx_cp(j + 1, 1 - b).start()

            @pl.when(j >= 2)
            def _():
                for r in range(4):
                    out_cp(j - 2, b, r).wait()

            ridx = [idx_v[pl.ds(b * BLK + g * 16, 16)] for g in range(8)]
            for r in range(4):
                for s in range(8):
                    d = 8 * r + s
                    row = buf_v.at[b * 4 + r, s]
                    for g in range(8):
                        v = plsc.load_gather(tbl_v, [ridx[g] + col_const[d]])
                        row[pl.ds(16 * g, 16)] = v

            for r in range(4):
                out_cp(j, b, r).start()

    for b in range(2):
        for r in range(4):
            out_cp(0, b, r).wait()


@functools.partial(jax.jit, static_argnames=())
def kernel(edge_type, table):
    idx = edge_type.astype(jnp.int32)
    tbl_t = table.T.reshape(-1)
    mesh = plsc.VectorSubcoreMesh(
        core_axis_name="c", subcore_axis_name="s", num_cores=NUM_CORES
    )
    run = pl.kernel(
        _body,
        out_type=jax.ShapeDtypeStruct((EMBED, N_EDGES), jnp.float32),
        mesh=mesh,
        scratch_types=[
            pltpu.VMEM((2 * BLK,), jnp.int32),
            pltpu.VMEM((EMBED * NUM_ROWS,), jnp.float32),
            pltpu.VMEM((8, 8, 128), jnp.float32),
            pltpu.SemaphoreType.DMA,
            pltpu.SemaphoreType.DMA((2,)),
            pltpu.SemaphoreType.DMA((2,)),
        ],
        compiler_params=pltpu.CompilerParams(needs_layout_passes=False),
    )
    out_t = run(idx, tbl_t)
    return out_t.T


# speedup vs baseline: 4.6218x; 1.9705x over previous
import functools

import jax
import jax.numpy as jnp
from jax import lax
from jax.experimental import pallas as pl
from jax.experimental.pallas import tpu as pltpu
from jax.experimental.pallas import tpu_sc as plsc

N_EDGES = 1600000
EMBED = 32
NUM_ROWS = 1000
NUM_CORES = 2
NUM_SUBCORES = 16
NUM_WORKERS = NUM_CORES * NUM_SUBCORES
BLK = 128
N_BLOCKS = N_EDGES // BLK
BASE_BLOCKS = N_BLOCKS // NUM_WORKERS
EXTRA = N_BLOCKS - BASE_BLOCKS * NUM_WORKERS
MAXBLK = BASE_BLOCKS + 1


def _body(idx_hbm, tbl_hbm, out_hbm, idx_v, tbl_v, buf_v, sem_t, sem_i, sem_o):
    wid = lax.axis_index("s") * NUM_CORES + lax.axis_index("c")
    nblk = jnp.where(wid < EXTRA, BASE_BLOCKS + 1, BASE_BLOCKS)
    blk0 = wid * BASE_BLOCKS + jnp.minimum(wid, EXTRA)

    pltpu.async_copy(tbl_hbm, tbl_v, sem_t).wait()

    col_const = [jnp.full((16,), d * NUM_ROWS, jnp.int32) for d in range(EMBED)]

    def idx_cp(j, b):
        return pltpu.make_async_copy(
            idx_hbm.at[pl.ds((blk0 + j) * BLK, BLK)],
            idx_v.at[pl.ds(b * BLK, BLK)], sem_i.at[b])

    def out_cp(j, b, r):
        return pltpu.make_async_copy(
            buf_v.at[b * 4 + r],
            out_hbm.at[pl.ds(8 * r, 8), pl.ds((blk0 + j) * BLK, BLK)],
            sem_o.at[b])

    idx_cp(0, 0).start()

    @pl.loop(0, MAXBLK)
    def step(j):
        b = lax.rem(j, 2)

        @pl.when(j < nblk)
        def _():
            idx_cp(j, b).wait()

            @pl.when(j + 1 < nblk)
            def _():
                idx_cp(j + 1, 1 - b).start()

            @pl.when(j >= 2)
            def _():
                for r in range(4):
                    out_cp(j - 2, b, r).wait()

            ridx = [idx_v[pl.ds(b * BLK + g * 16, 16)] for g in range(8)]
            for r in range(4):
                for s in range(8):
                    d = 8 * r + s
                    row = buf_v.at[b * 4 + r, s]
                    vs = [plsc.load_gather(tbl_v, [ridx[g] + col_const[d]])
                          for g in range(8)]
                    for g in range(8):
                        row[pl.ds(16 * g, 16)] = vs[g]

            for r in range(4):
                out_cp(j, b, r).start()

    for b in range(2):
        for r in range(4):
            out_cp(0, b, r).wait()


@functools.partial(jax.jit, static_argnames=())
def kernel(edge_type, table):
    idx = edge_type.astype(jnp.int32)
    tbl_t = table.T.reshape(-1)
    mesh = plsc.VectorSubcoreMesh(
        core_axis_name="c", subcore_axis_name="s", num_cores=NUM_CORES
    )
    run = pl.kernel(
        _body,
        out_type=jax.ShapeDtypeStruct((EMBED, N_EDGES), jnp.float32),
        mesh=mesh,
        scratch_types=[
            pltpu.VMEM((2 * BLK,), jnp.int32),
            pltpu.VMEM((EMBED * NUM_ROWS,), jnp.float32),
            pltpu.VMEM((8, 8, 128), jnp.float32),
            pltpu.SemaphoreType.DMA,
            pltpu.SemaphoreType.DMA((2,)),
            pltpu.SemaphoreType.DMA((2,)),
        ],
        compiler_params=pltpu.CompilerParams(needs_layout_passes=False),
    )
    out_t = run(idx, tbl_t)
    return out_t.T
